# baseline (speedup 1.0000x reference)
import functools

import jax
import jax.numpy as jnp
from jax import lax
from jax.experimental import pallas as pl
from jax.experimental.pallas import tpu as pltpu
from jax.experimental.pallas import tpu_sc as plsc

OUT_DIM = 128
N_MONTH, N_WEEK, N_DAY = 13, 6, 32
N_COMBO = N_MONTH * N_WEEK * N_DAY

TOTAL = 1024 * 200
NC, NS = 2, 16
NW = NC * NS
PER_W = TOTAL // NW
CHUNK = 128
NCH = PER_W // CHUNK
IDX_ROWS = TOTAL // CHUNK
NPAIR = NCH // 2
NBUF = 3
LEAD = 2


_PREP_GRID = 5
_PREP_ROWS = 200 // _PREP_GRID


def _tc_prep_body(m_ref, w_ref, d_ref, mt_ref, wt_ref, dt_ref,
                  table_ref, cidx_ref):
    cidx_ref[...] = (m_ref[...] * (N_WEEK * N_DAY)
                     + w_ref[...] * N_DAY + d_ref[...])

    @pl.when(pl.program_id(0) == 0)
    def _():
        r = lax.broadcasted_iota(jnp.int32, (N_COMBO, N_DAY), 0)
        c = lax.broadcasted_iota(jnp.int32, (N_COMBO, N_DAY), 1)
        acc = jnp.dot((c == r % N_DAY).astype(jnp.float32), dt_ref[...],
                      preferred_element_type=jnp.float32)
        r = lax.broadcasted_iota(jnp.int32, (N_COMBO, N_WEEK), 0)
        c = lax.broadcasted_iota(jnp.int32, (N_COMBO, N_WEEK), 1)
        acc += jnp.dot((c == (r // N_DAY) % N_WEEK).astype(jnp.float32),
                       wt_ref[...], preferred_element_type=jnp.float32)
        r = lax.broadcasted_iota(jnp.int32, (N_COMBO, N_MONTH), 0)
        c = lax.broadcasted_iota(jnp.int32, (N_COMBO, N_MONTH), 1)
        acc += jnp.dot((c == r // (N_WEEK * N_DAY)).astype(jnp.float32),
                       mt_ref[...], preferred_element_type=jnp.float32)
        table_ref[...] = acc


_tc_prep = pl.pallas_call(
    _tc_prep_body,
    grid=(_PREP_GRID,),
    in_specs=[
        pl.BlockSpec((_PREP_ROWS, 1024), lambda i: (i, 0)),
        pl.BlockSpec((_PREP_ROWS, 1024), lambda i: (i, 0)),
        pl.BlockSpec((_PREP_ROWS, 1024), lambda i: (i, 0)),
        pl.BlockSpec((N_MONTH, OUT_DIM), lambda i: (0, 0)),
        pl.BlockSpec((N_WEEK, OUT_DIM), lambda i: (0, 0)),
        pl.BlockSpec((N_DAY, OUT_DIM), lambda i: (0, 0)),
    ],
    out_specs=[
        pl.BlockSpec((N_COMBO, OUT_DIM), lambda i: (0, 0)),
        pl.BlockSpec((_PREP_ROWS, 1024), lambda i: (i, 0)),
    ],
    out_shape=[
        jax.ShapeDtypeStruct((N_COMBO, OUT_DIM), jnp.float32),
        jax.ShapeDtypeStruct((200, 1024), jnp.int32),
    ],
)


def _sc_gather_body(table_hbm, cidx_hbm, out_hbm, idx_v, tab_sp, rows,
                    gsem, ssem):
    cid = lax.axis_index("c")
    sid = lax.axis_index("s")
    wid = sid * NC + cid

    idx_cp = pltpu.async_copy(cidx_hbm.at[wid], idx_v, ssem[0])

    def gather_pair(j, b, src):
        c0 = pltpu.async_copy(src.at[idx_v.at[2 * j]],
                              rows[b].at[pl.ds(0, CHUNK)], gsem[b])
        c1 = pltpu.async_copy(src.at[idx_v.at[2 * j + 1]],
                              rows[b].at[pl.ds(CHUNK, CHUNK)], gsem[b])
        return (c0, c1)

    def scatter_pair(j, b):
        base = wid * PER_W + j * (2 * CHUNK)
        return pltpu.async_copy(rows[b], out_hbm.at[pl.ds(base, 2 * CHUNK)],
                                ssem[b])

    @pl.when(sid < N_COMBO // 192)
    def _():
        sl = pl.ds(sid * 192, 192)
        pltpu.sync_copy(table_hbm.at[sl], tab_sp.at[sl])

    idx_cp.wait()
    plsc.subcore_barrier()

    g = [None] * NBUF
    s = [None] * NBUF
    for j in range(LEAD):
        g[j % NBUF] = gather_pair(j, j % NBUF, tab_sp)
    for j in range(NPAIR):
        nj = j + LEAD
        if nj < NPAIR:
            nb = nj % NBUF
            if s[nb] is not None:
                s[nb].wait()
            g[nb] = gather_pair(nj, nb, tab_sp)
        b = j % NBUF
        g[b][0].wait()
        g[b][1].wait()
        s[b] = scatter_pair(j, b)
    for j in range(max(NPAIR - NBUF, 0), NPAIR):
        s[j % NBUF].wait()


_sc_gather = functools.partial(
    pl.kernel,
    out_type=jax.ShapeDtypeStruct((TOTAL, OUT_DIM), jnp.float32),
    mesh=plsc.VectorSubcoreMesh(core_axis_name="c", subcore_axis_name="s",
                                num_cores=NC, num_subcores=NS),
    scratch_types=[
        pltpu.VMEM((NCH, CHUNK), jnp.int32),
        pltpu.VMEM_SHARED((N_COMBO, OUT_DIM), jnp.float32),
        [pltpu.VMEM((2 * CHUNK, OUT_DIM), jnp.float32) for _ in range(NBUF)],
        [pltpu.SemaphoreType.DMA for _ in range(NBUF)],
        [pltpu.SemaphoreType.DMA for _ in range(NBUF)],
    ],
)(_sc_gather_body)


def kernel(month, weekday, day, day_table, week_table, month_table):
    m = month.astype(jnp.int32).T
    w = weekday.astype(jnp.int32).T
    d = day.astype(jnp.int32).T
    table, cidx_t = _tc_prep(m, w, d, month_table, week_table, day_table)
    cidx = lax.reshape(cidx_t, (NW, NCH, CHUNK), dimensions=(1, 0))
    out = _sc_gather(table, cidx)
    return out.reshape(month.shape + (OUT_DIM,))


# speedup vs baseline: 1.0258x; 1.0258x over previous
import functools

import jax
import jax.numpy as jnp
from jax import lax
from jax.experimental import pallas as pl
from jax.experimental.pallas import tpu as pltpu
from jax.experimental.pallas import tpu_sc as plsc

OUT_DIM = 128
N_MONTH, N_WEEK, N_DAY = 13, 6, 32
N_COMBO = N_MONTH * N_WEEK * N_DAY

TOTAL = 1024 * 200
NC, NS = 2, 16
NW = NC * NS
PER_W = TOTAL // NW
CHUNK = 128
NCH = PER_W // CHUNK
IDX_ROWS = TOTAL // CHUNK
NPAIR = NCH // 2
NBUF = 3
LEAD = 2


def _tc_prep_body(m_ref, w_ref, d_ref, mt_ref, wt_ref, dt_ref,
                  table_ref, cidx_ref):
    cidx_ref[...] = (m_ref[...] * (N_WEEK * N_DAY)
                     + w_ref[...] * N_DAY + d_ref[...])

    r = lax.broadcasted_iota(jnp.int32, (N_COMBO, N_DAY), 0)
    c = lax.broadcasted_iota(jnp.int32, (N_COMBO, N_DAY), 1)
    acc = jnp.dot((c == r % N_DAY).astype(jnp.float32), dt_ref[...],
                  preferred_element_type=jnp.float32)
    r = lax.broadcasted_iota(jnp.int32, (N_COMBO, N_WEEK), 0)
    c = lax.broadcasted_iota(jnp.int32, (N_COMBO, N_WEEK), 1)
    acc += jnp.dot((c == (r // N_DAY) % N_WEEK).astype(jnp.float32),
                   wt_ref[...], preferred_element_type=jnp.float32)
    r = lax.broadcasted_iota(jnp.int32, (N_COMBO, N_MONTH), 0)
    c = lax.broadcasted_iota(jnp.int32, (N_COMBO, N_MONTH), 1)
    acc += jnp.dot((c == r // (N_WEEK * N_DAY)).astype(jnp.float32),
                   mt_ref[...], preferred_element_type=jnp.float32)
    table_ref[...] = acc


_tc_prep = pl.pallas_call(
    _tc_prep_body,
    out_shape=[
        jax.ShapeDtypeStruct((N_COMBO, OUT_DIM), jnp.float32),
        jax.ShapeDtypeStruct((200, 1024), jnp.int32),
    ],
)


def _sc_gather_body(table_hbm, cidx_hbm, out_hbm, idx_v, tab_sp, rows,
                    gsem, ssem):
    cid = lax.axis_index("c")
    sid = lax.axis_index("s")
    wid = sid * NC + cid

    idx_cp = pltpu.async_copy(cidx_hbm.at[wid], idx_v, ssem[0])

    def gather_pair(j, b, src):
        c0 = pltpu.async_copy(src.at[idx_v.at[2 * j]],
                              rows[b].at[pl.ds(0, CHUNK)], gsem[b])
        c1 = pltpu.async_copy(src.at[idx_v.at[2 * j + 1]],
                              rows[b].at[pl.ds(CHUNK, CHUNK)], gsem[b])
        return (c0, c1)

    def scatter_pair(j, b):
        base = wid * PER_W + j * (2 * CHUNK)
        return pltpu.async_copy(rows[b], out_hbm.at[pl.ds(base, 2 * CHUNK)],
                                ssem[b])

    @pl.when(sid < N_COMBO // 192)
    def _():
        sl = pl.ds(sid * 192, 192)
        pltpu.sync_copy(table_hbm.at[sl], tab_sp.at[sl])

    idx_cp.wait()
    plsc.subcore_barrier()

    g = [None] * NBUF
    s = [None] * NBUF
    for j in range(LEAD):
        g[j % NBUF] = gather_pair(j, j % NBUF, tab_sp)
    for j in range(NPAIR):
        nj = j + LEAD
        if nj < NPAIR:
            nb = nj % NBUF
            if s[nb] is not None:
                s[nb].wait()
            g[nb] = gather_pair(nj, nb, tab_sp)
        b = j % NBUF
        g[b][0].wait()
        g[b][1].wait()
        s[b] = scatter_pair(j, b)
    for j in range(max(NPAIR - NBUF, 0), NPAIR):
        s[j % NBUF].wait()


_sc_gather = functools.partial(
    pl.kernel,
    out_type=jax.ShapeDtypeStruct((TOTAL, OUT_DIM), jnp.float32),
    mesh=plsc.VectorSubcoreMesh(core_axis_name="c", subcore_axis_name="s",
                                num_cores=NC, num_subcores=NS),
    scratch_types=[
        pltpu.VMEM((NCH, CHUNK), jnp.int32),
        pltpu.VMEM_SHARED((N_COMBO, OUT_DIM), jnp.float32),
        [pltpu.VMEM((2 * CHUNK, OUT_DIM), jnp.float32) for _ in range(NBUF)],
        [pltpu.SemaphoreType.DMA for _ in range(NBUF)],
        [pltpu.SemaphoreType.DMA for _ in range(NBUF)],
    ],
)(_sc_gather_body)


def kernel(month, weekday, day, day_table, week_table, month_table):
    m = month.astype(jnp.int32).T
    w = weekday.astype(jnp.int32).T
    d = day.astype(jnp.int32).T
    table, cidx_t = _tc_prep(m, w, d, month_table, week_table, day_table)
    cidx = lax.reshape(cidx_t, (NW, NCH, CHUNK), dimensions=(1, 0))
    out = _sc_gather(table, cidx)
    return out.reshape(month.shape + (OUT_DIM,))


# speedup vs baseline: 1.0312x; 1.0053x over previous
import functools

import jax
import jax.numpy as jnp
from jax import lax
from jax.experimental import pallas as pl
from jax.experimental.pallas import tpu as pltpu
from jax.experimental.pallas import tpu_sc as plsc

OUT_DIM = 128
N_MONTH, N_WEEK, N_DAY = 13, 6, 32
N_COMBO = N_MONTH * N_WEEK * N_DAY

TOTAL = 1024 * 200
NC, NS = 2, 16
NW = NC * NS
PER_W = TOTAL // NW
CHUNK = 128
NCH = PER_W // CHUNK
NPAIR = NCH // 2
NBUF = 3
LEAD = 2


def _tc_prep_body(m_ref, w_ref, d_ref, mt_ref, wt_ref, dt_ref,
                  table_ref, cidx_ref):
    cidx_ref[...] = (m_ref[...] * (N_WEEK * N_DAY)
                     + w_ref[...] * N_DAY + d_ref[...])

    r = lax.broadcasted_iota(jnp.int32, (N_COMBO, N_DAY), 0)
    c = lax.broadcasted_iota(jnp.int32, (N_COMBO, N_DAY), 1)
    acc = jnp.dot((c == r % N_DAY).astype(jnp.float32), dt_ref[...],
                  preferred_element_type=jnp.float32)
    r = lax.broadcasted_iota(jnp.int32, (N_COMBO, N_WEEK), 0)
    c = lax.broadcasted_iota(jnp.int32, (N_COMBO, N_WEEK), 1)
    acc += jnp.dot((c == (r // N_DAY) % N_WEEK).astype(jnp.float32),
                   wt_ref[...], preferred_element_type=jnp.float32)
    r = lax.broadcasted_iota(jnp.int32, (N_COMBO, N_MONTH), 0)
    c = lax.broadcasted_iota(jnp.int32, (N_COMBO, N_MONTH), 1)
    acc += jnp.dot((c == r // (N_WEEK * N_DAY)).astype(jnp.float32),
                   mt_ref[...], preferred_element_type=jnp.float32)
    table_ref[...] = acc


_tc_prep = pl.pallas_call(
    _tc_prep_body,
    out_shape=[
        jax.ShapeDtypeStruct((N_COMBO, OUT_DIM), jnp.float32),
        jax.ShapeDtypeStruct((200, 1024), jnp.int32),
    ],
)


def _sc_gather_body(table_hbm, cidx_hbm, out_hbm, idx_v, tab_sp, rows,
                    gsem, ssem):
    cid = lax.axis_index("c")
    sid = lax.axis_index("s")
    wid = sid * NC + cid

    idx_cp = pltpu.async_copy(cidx_hbm.at[wid], idx_v, ssem[0])

    def gather_pair(j, b, src):
        c0 = pltpu.async_copy(src.at[idx_v.at[2 * j]],
                              rows[b].at[pl.ds(0, CHUNK)], gsem[b])
        c1 = pltpu.async_copy(src.at[idx_v.at[2 * j + 1]],
                              rows[b].at[pl.ds(CHUNK, CHUNK)], gsem[b])
        return (c0, c1)

    def scatter_pair(j, b):
        base = wid * PER_W + j * (2 * CHUNK)
        return pltpu.async_copy(rows[b], out_hbm.at[pl.ds(base, 2 * CHUNK)],
                                ssem[b])

    @pl.when(sid < N_COMBO // 192)
    def _():
        sl = pl.ds(sid * 192, 192)
        pltpu.sync_copy(table_hbm.at[sl], tab_sp.at[sl])

    idx_cp.wait()
    plsc.subcore_barrier()

    g = [None] * NBUF
    s = [None] * NBUF
    for j in range(LEAD):
        g[j % NBUF] = gather_pair(j, j % NBUF, tab_sp)
    for j in range(NPAIR):
        nj = j + LEAD
        if nj < NPAIR:
            nb = nj % NBUF
            if s[nb] is not None:
                s[nb].wait()
            g[nb] = gather_pair(nj, nb, tab_sp)
        b = j % NBUF
        g[b][0].wait()
        g[b][1].wait()
        s[b] = scatter_pair(j, b)
    for j in range(max(NPAIR - NBUF, 0), NPAIR):
        s[j % NBUF].wait()


_sc_gather = functools.partial(
    pl.kernel,
    out_type=jax.ShapeDtypeStruct((TOTAL, OUT_DIM), jnp.float32),
    mesh=plsc.VectorSubcoreMesh(core_axis_name="c", subcore_axis_name="s",
                                num_cores=NC, num_subcores=NS),
    scratch_types=[
        pltpu.VMEM((NCH, CHUNK), jnp.int32),
        pltpu.VMEM_SHARED((N_COMBO, OUT_DIM), jnp.float32),
        [pltpu.VMEM((2 * CHUNK, OUT_DIM), jnp.float32) for _ in range(NBUF)],
        [pltpu.SemaphoreType.DMA for _ in range(NBUF)],
        [pltpu.SemaphoreType.DMA for _ in range(NBUF)],
    ],
)(_sc_gather_body)


def kernel(month, weekday, day, day_table, week_table, month_table):
    m = month.astype(jnp.int32).T
    w = weekday.astype(jnp.int32).T
    d = day.astype(jnp.int32).T
    table, cidx_t = _tc_prep(m, w, d, month_table, week_table, day_table)
    cidx = lax.reshape(cidx_t, (NW, NCH, CHUNK), dimensions=(1, 0))
    out = _sc_gather(table, cidx)
    return out.reshape(month.shape + (OUT_DIM,))
